# baseline (speedup 1.0000x reference)
import functools

import jax
import jax.numpy as jnp
from jax import lax
from jax.experimental import pallas as pl
from jax.experimental.pallas import tpu as pltpu

IC = 128
NGROUP = 32
CPG = IC // NGROUP
EPSV = 1e-5
NSEG = 16


def _dot_t(a, b):
    return lax.dot_general(a, b, (((0,), (0,)), ((), ())),
                           preferred_element_type=jnp.float32)


def _onehot(bid_col, rows):
    seg = lax.broadcasted_iota(jnp.int32, (rows, NSEG), 1)
    return (bid_col == seg).astype(jnp.float32)


def _body(nblocks, rows, x_ref, bid_ref, w_ref, b_ref, o_ref,
          xs, s1, s2, cnt):
    p = pl.program_id(0)
    j = pl.program_id(1)

    @pl.when((p == 0) & (j == 0))
    def _():
        s1[...] = jnp.zeros_like(s1)
        s2[...] = jnp.zeros_like(s2)
        cnt[...] = jnp.zeros_like(cnt)

    @pl.when(p == 0)
    def _():
        x = x_ref[...]
        xs[pl.ds(j * rows, rows), :] = x
        oh = _onehot(bid_ref[...], rows)
        s1[...] += _dot_t(oh, x)
        s2[...] += _dot_t(oh, x * x)
        cnt[...] += _dot_t(oh, jnp.ones_like(x))

        @pl.when(j == nblocks - 1)
        def _():
            ic = 1.0 / (cnt[...] * CPG + EPSV)
            ci = lax.broadcasted_iota(jnp.int32, (IC, IC), 0) // CPG
            cj = lax.broadcasted_iota(jnp.int32, (IC, IC), 1) // CPG
            ggt = (ci == cj).astype(jnp.float32)
            a1 = lax.dot_general(s1[...], ggt, (((1,), (0,)), ((), ())),
                                 preferred_element_type=jnp.float32)
            a2 = lax.dot_general(s2[...], ggt, (((1,), (0,)), ((), ())),
                                 preferred_element_type=jnp.float32)
            mg = a1 * ic
            var = ic * (a2 - 2.0 * mg * a1 + cnt[...] * CPG * mg * mg)
            istd = lax.rsqrt(var + EPSV)
            w = w_ref[...]
            scale = istd * w
            shift = b_ref[...] - mg * scale
            s1[...] = scale
            s2[...] = shift

    @pl.when(p == 1)
    def _():
        x = xs[pl.ds(j * rows, rows), :]
        oh = _onehot(bid_ref[...], rows)
        rs = lax.dot_general(oh, s1[...], (((1,), (0,)), ((), ())),
                             preferred_element_type=jnp.float32)
        rh = lax.dot_general(oh, s2[...], (((1,), (0,)), ((), ())),
                             preferred_element_type=jnp.float32)
        o_ref[...] = x * rs + rh


def kernel(data, batch_id, batch_size, weights, bias):
    n, c = data.shape
    rows = 2000
    nblocks = n // rows
    assert nblocks * rows == n
    bid_col = batch_id.astype(jnp.int32).reshape(n, 1)

    out = pl.pallas_call(
        functools.partial(_body, nblocks, rows),
        grid=(2, nblocks),
        in_specs=[
            pl.BlockSpec((rows, c), lambda p, j: (jnp.where(p == 0, j, 0), 0)),
            pl.BlockSpec((rows, 1), lambda p, j: (j, 0)),
            pl.BlockSpec((1, c), lambda p, j: (0, 0)),
            pl.BlockSpec((1, c), lambda p, j: (0, 0)),
        ],
        out_specs=pl.BlockSpec((rows, c),
                               lambda p, j: (jnp.where(p == 0, 0, j), 0)),
        out_shape=jax.ShapeDtypeStruct((n, c), jnp.float32),
        scratch_shapes=[
            pltpu.VMEM((n, c), jnp.float32),
            pltpu.VMEM((NSEG, c), jnp.float32),
            pltpu.VMEM((NSEG, c), jnp.float32),
            pltpu.VMEM((NSEG, c), jnp.float32),
        ],
        compiler_params=pltpu.CompilerParams(
            dimension_semantics=("arbitrary", "arbitrary")),
    )(data, bid_col, weights, bias)
    return out


# speedup vs baseline: 1.7868x; 1.7868x over previous
import functools

import jax
import jax.numpy as jnp
from jax import lax
from jax.experimental import pallas as pl
from jax.experimental.pallas import tpu as pltpu

IC = 128
NGROUP = 32
CPG = IC // NGROUP
EPSV = 1e-5
NSEG = 16


def _onehot_t(bid_row, rows):
    seg = lax.broadcasted_iota(jnp.int32, (NSEG, rows), 0)
    return (bid_row[None, :] == seg).astype(jnp.float32)


def _body(nblocks, rows, x_ref, bid_ref, w_ref, b_ref, o_ref,
          xs, s1, s2, cnt):
    p = pl.program_id(0)
    j = pl.program_id(1)

    @pl.when((p == 0) & (j == 0))
    def _():
        s1[...] = jnp.zeros_like(s1)
        s2[...] = jnp.zeros_like(s2)
        cnt[...] = jnp.zeros_like(cnt)

    @pl.when(p == 0)
    def _():
        x = x_ref[...]
        xs[pl.ds(j * rows, rows), :] = x
        oht = _onehot_t(bid_ref[0, 0, :], rows)
        s1[...] += lax.dot_general(oht, x, (((1,), (0,)), ((), ())),
                                   preferred_element_type=jnp.float32)
        s2[...] += lax.dot_general(oht, x * x, (((1,), (0,)), ((), ())),
                                   preferred_element_type=jnp.float32)
        cnt[...] += lax.dot_general(oht, jnp.ones_like(x),
                                    (((1,), (0,)), ((), ())),
                                    preferred_element_type=jnp.float32)

        @pl.when(j == nblocks - 1)
        def _():
            ic = 1.0 / (cnt[...] * CPG + EPSV)
            ci = lax.broadcasted_iota(jnp.int32, (IC, IC), 0) // CPG
            cj = lax.broadcasted_iota(jnp.int32, (IC, IC), 1) // CPG
            ggt = (ci == cj).astype(jnp.float32)
            a1 = lax.dot_general(s1[...], ggt, (((1,), (0,)), ((), ())),
                                 preferred_element_type=jnp.float32)
            a2 = lax.dot_general(s2[...], ggt, (((1,), (0,)), ((), ())),
                                 preferred_element_type=jnp.float32)
            mg = a1 * ic
            var = ic * (a2 - 2.0 * mg * a1 + cnt[...] * CPG * mg * mg)
            istd = lax.rsqrt(var + EPSV)
            w = w_ref[...]
            scale = istd * w
            shift = b_ref[...] - mg * scale
            s1[...] = scale
            s2[...] = shift

    @pl.when(p == 1)
    def _():
        x = xs[pl.ds(j * rows, rows), :]
        oht = _onehot_t(bid_ref[0, 0, :], rows)
        rs = lax.dot_general(oht, s1[...], (((0,), (0,)), ((), ())),
                             preferred_element_type=jnp.float32)
        rh = lax.dot_general(oht, s2[...], (((0,), (0,)), ((), ())),
                             preferred_element_type=jnp.float32)
        o_ref[...] = x * rs + rh


def kernel(data, batch_id, batch_size, weights, bias):
    n, c = data.shape
    rows = 2000
    nblocks = n // rows
    assert nblocks * rows == n
    bid3 = batch_id.astype(jnp.int32).reshape(nblocks, 1, rows)

    out = pl.pallas_call(
        functools.partial(_body, nblocks, rows),
        grid=(2, nblocks),
        in_specs=[
            pl.BlockSpec((rows, c), lambda p, j: (jnp.where(p == 0, j, 0), 0)),
            pl.BlockSpec((1, 1, rows), lambda p, j: (j, 0, 0)),
            pl.BlockSpec((1, c), lambda p, j: (0, 0)),
            pl.BlockSpec((1, c), lambda p, j: (0, 0)),
        ],
        out_specs=pl.BlockSpec((rows, c),
                               lambda p, j: (jnp.where(p == 0, 0, j), 0)),
        out_shape=jax.ShapeDtypeStruct((n, c), jnp.float32),
        scratch_shapes=[
            pltpu.VMEM((n, c), jnp.float32),
            pltpu.VMEM((NSEG, c), jnp.float32),
            pltpu.VMEM((NSEG, c), jnp.float32),
            pltpu.VMEM((NSEG, c), jnp.float32),
        ],
        compiler_params=pltpu.CompilerParams(
            dimension_semantics=("arbitrary", "arbitrary")),
    )(data, bid3, weights, bias)
    return out


# speedup vs baseline: 1.8179x; 1.0174x over previous
import functools

import jax
import jax.numpy as jnp
from jax import lax
from jax.experimental import pallas as pl
from jax.experimental.pallas import tpu as pltpu

IC = 128
NGROUP = 32
CPG = IC // NGROUP
EPSV = 1e-5
NSEG = 16


def _seg_mask(bid_row, rows):
    seg = lax.broadcasted_iota(jnp.int32, (NSEG, rows), 0)
    return bid_row[None, :] == seg


def _body(nblocks, rows, x_ref, bid_ref, w_ref, b_ref, o_ref,
          xs, s1, s2, cnt, tab):
    p = pl.program_id(0)
    j = pl.program_id(1)

    @pl.when((p == 0) & (j == 0))
    def _():
        s1[...] = jnp.zeros_like(s1)
        s2[...] = jnp.zeros_like(s2)
        cnt[...] = jnp.zeros_like(cnt)

    @pl.when(p == 0)
    def _():
        x = x_ref[...]
        xs[pl.ds(j * rows, rows), :] = x
        mask = _seg_mask(bid_ref[0, 0, :], rows)
        oht = mask.astype(jnp.bfloat16)
        xb = x.astype(jnp.bfloat16)
        s1[...] += lax.dot_general(oht, xb, (((1,), (0,)), ((), ())),
                                   preferred_element_type=jnp.float32)
        s2[...] += lax.dot_general(oht, xb * xb, (((1,), (0,)), ((), ())),
                                   preferred_element_type=jnp.float32)
        cnt[...] += jnp.sum(mask.astype(jnp.float32), axis=1)[:, None]

        @pl.when(j == nblocks - 1)
        def _():
            ic = 1.0 / (cnt[...] * CPG + EPSV)
            ci = lax.broadcasted_iota(jnp.int32, (IC, IC), 0) // CPG
            cj = lax.broadcasted_iota(jnp.int32, (IC, IC), 1) // CPG
            ggt = (ci == cj).astype(jnp.float32)
            a1 = lax.dot_general(s1[...], ggt, (((1,), (0,)), ((), ())),
                                 preferred_element_type=jnp.float32)
            a2 = lax.dot_general(s2[...], ggt, (((1,), (0,)), ((), ())),
                                 preferred_element_type=jnp.float32)
            mg = a1 * ic
            var = ic * (a2 - 2.0 * mg * a1 + cnt[...] * CPG * mg * mg)
            istd = lax.rsqrt(var + EPSV)
            w = w_ref[...]
            scale = istd * w
            shift = b_ref[...] - mg * scale
            tab[...] = jnp.concatenate([scale, shift],
                                       axis=1).astype(jnp.bfloat16)

    @pl.when(p == 1)
    def _():
        x = xs[pl.ds(j * rows, rows), :]
        oht = _seg_mask(bid_ref[0, 0, :], rows).astype(jnp.bfloat16)
        rsh = lax.dot_general(oht, tab[...], (((0,), (0,)), ((), ())),
                              preferred_element_type=jnp.float32)
        o_ref[...] = x * rsh[:, :IC] + rsh[:, IC:]


def kernel(data, batch_id, batch_size, weights, bias):
    n, c = data.shape
    rows = 2000
    nblocks = n // rows
    assert nblocks * rows == n
    bid3 = batch_id.astype(jnp.int32).reshape(nblocks, 1, rows)

    out = pl.pallas_call(
        functools.partial(_body, nblocks, rows),
        grid=(2, nblocks),
        in_specs=[
            pl.BlockSpec((rows, c), lambda p, j: (jnp.where(p == 0, j, 0), 0)),
            pl.BlockSpec((1, 1, rows), lambda p, j: (j, 0, 0)),
            pl.BlockSpec((1, c), lambda p, j: (0, 0)),
            pl.BlockSpec((1, c), lambda p, j: (0, 0)),
        ],
        out_specs=pl.BlockSpec((rows, c),
                               lambda p, j: (jnp.where(p == 0, 0, j), 0)),
        out_shape=jax.ShapeDtypeStruct((n, c), jnp.float32),
        scratch_shapes=[
            pltpu.VMEM((n, c), jnp.float32),
            pltpu.VMEM((NSEG, c), jnp.float32),
            pltpu.VMEM((NSEG, c), jnp.float32),
            pltpu.VMEM((NSEG, c), jnp.float32),
            pltpu.VMEM((NSEG, 2 * c), jnp.bfloat16),
        ],
        compiler_params=pltpu.CompilerParams(
            dimension_semantics=("arbitrary", "arbitrary")),
    )(data, bid3, weights, bias)
    return out
